# baseline (speedup 1.0000x reference)
import functools

import jax
import jax.numpy as jnp
from jax import lax
from jax.experimental import pallas as pl
from jax.experimental.pallas import tpu as pltpu
from jax.experimental.pallas import tpu_sc as plsc

VOCAB = 100000
HIDDEN = 64
B = 4096
L = 200

_NC = 2
_NS = 16
_NW = _NC * _NS

_RPC = 8
_SUB = 2
_ROWS_PER_W = B // _NW
_CHUNKS_PER_W = _ROWS_PER_W // _RPC
_OFFS = list(range(0, L - 16, 16)) + [L - 16]


def _sc_emb_body(x_hbm, tok_hbm, pos_hbm, out_hbm,
                 pos_v, xi_v, rows_v, out_v, sem):
    wid = lax.axis_index("s") * _NC + lax.axis_index("c")
    pltpu.sync_copy(pos_hbm, pos_v)
    row0 = wid * _ROWS_PER_W

    def chunk_body(ch, carry):
        r0 = row0 + ch * _RPC
        pltpu.sync_copy(x_hbm.at[pl.ds(r0, _RPC)], xi_v)

        def sub_body(r, c1):
            cps = []
            for b in range(_SUB):
                for off in _OFFS:
                    idx16 = xi_v[_SUB * r + b, pl.ds(off, 16)]
                    cps.append(pltpu.async_copy(
                        tok_hbm.at[idx16],
                        rows_v.at[pl.ds(b * L + off, 16)],
                        sem,
                    ))
            for cp in cps:
                cp.wait()

            def row_body(l, c2):
                for b in range(_SUB):
                    for c in range(HIDDEN // 16):
                        sl = pl.ds(c * 16, 16)
                        out_v[b, l, sl] = rows_v[b * L + l, sl] + pos_v[l, sl]
                return c2

            lax.fori_loop(0, L, row_body, 0)
            pltpu.sync_copy(out_v, out_hbm.at[pl.ds(r0 + _SUB * r, _SUB)])
            return c1

        lax.fori_loop(0, _RPC // _SUB, sub_body, 0)
        return carry

    lax.fori_loop(0, _CHUNKS_PER_W, chunk_body, 0)


def _sc_emb(x, tok_wide, pos_table):
    mesh = plsc.VectorSubcoreMesh(core_axis_name="c", subcore_axis_name="s")
    f = functools.partial(
        pl.kernel,
        mesh=mesh,
        out_type=jax.ShapeDtypeStruct((B, L, HIDDEN), jnp.float32),
        scratch_types=[
            pltpu.VMEM((L, HIDDEN), jnp.float32),
            pltpu.VMEM((_RPC, L), jnp.int32),
            pltpu.VMEM((_SUB * L, 128), jnp.float32),
            pltpu.VMEM((_SUB, L, HIDDEN), jnp.float32),
            pltpu.SemaphoreType.DMA,
        ],
        compiler_params=pltpu.CompilerParams(use_tc_tiling_on_sc=True),
    )(_sc_emb_body)
    return f(x, tok_wide, pos_table)


_RPB = 1000


def _repack_body(t_ref, o_ref):
    o_ref[:, :HIDDEN] = t_ref[...]


def _tc_repack(token_table):
    return pl.pallas_call(
        _repack_body,
        grid=(VOCAB // _RPB,),
        in_specs=[pl.BlockSpec((_RPB, HIDDEN), lambda i: (i, 0))],
        out_specs=pl.BlockSpec((_RPB, 128), lambda i: (i, 0)),
        out_shape=jax.ShapeDtypeStruct((VOCAB, 128), jnp.float32),
    )(token_table)


_MASK_BB = 128


def _mask_body(x_ref, o_ref):
    m = x_ref[...] > 0
    o_ref[...] = jnp.broadcast_to(m[:, None, :], (_MASK_BB, L, L))


def _tc_mask(x):
    return pl.pallas_call(
        _mask_body,
        grid=(B // _MASK_BB,),
        in_specs=[pl.BlockSpec((_MASK_BB, L), lambda i: (i, 0))],
        out_specs=pl.BlockSpec((_MASK_BB, L, L), lambda i: (i, 0, 0)),
        out_shape=jax.ShapeDtypeStruct((B, L, L), jnp.bool_),
    )(x)


def kernel(x, token_table, pos_table):
    tok_wide = _tc_repack(token_table)
    emb = _sc_emb(x, tok_wide, pos_table)
    mask = _tc_mask(x).reshape(B, 1, L, L)
    return emb, mask


# speedup vs baseline: 1.8151x; 1.8151x over previous
import functools

import jax
import jax.numpy as jnp
from jax import lax
from jax.experimental import pallas as pl
from jax.experimental.pallas import tpu as pltpu
from jax.experimental.pallas import tpu_sc as plsc

VOCAB = 100000
HIDDEN = 64
B = 4096
L = 200

_NC = 2
_NS = 16
_NW = _NC * _NS

_RPC = 8
_SUB = 2
_ROWS_PER_W = B // _NW
_CHUNKS_PER_W = _ROWS_PER_W // _RPC
_OFFS = list(range(0, L - 16, 16)) + [L - 16]


def _sc_emb_body(x_hbm, tok_hbm, pos_hbm, out_hbm,
                 pos_v, xi_v, rows_v, out_v, sem):
    wid = lax.axis_index("s") * _NC + lax.axis_index("c")
    pltpu.sync_copy(pos_hbm, pos_v)
    row0 = wid * _ROWS_PER_W

    def chunk_body(ch, carry):
        r0 = row0 + ch * _RPC
        pltpu.sync_copy(x_hbm.at[pl.ds(r0, _RPC)], xi_v)

        def sub_body(r, c1):
            cps = []
            for b in range(_SUB):
                for off in _OFFS:
                    idx16 = xi_v[_SUB * r + b, pl.ds(off, 16)]
                    cps.append(pltpu.async_copy(
                        tok_hbm.at[idx16],
                        rows_v.at[pl.ds(b * L + off, 16)],
                        sem,
                    ))
            for cp in cps:
                cp.wait()

            def row_body(l, c2):
                for b in range(_SUB):
                    for c in range(HIDDEN // 16):
                        sl = pl.ds(c * 16, 16)
                        out_v[b, l, sl] = rows_v[b * L + l, sl] + pos_v[l, sl]
                return c2

            lax.fori_loop(0, L, row_body, 0)
            pltpu.sync_copy(out_v, out_hbm.at[pl.ds(r0 + _SUB * r, _SUB)])
            return c1

        lax.fori_loop(0, _RPC // _SUB, sub_body, 0)
        return carry

    lax.fori_loop(0, _CHUNKS_PER_W, chunk_body, 0)


def _sc_emb(x, tok_wide, pos_table):
    mesh = plsc.VectorSubcoreMesh(core_axis_name="c", subcore_axis_name="s")
    f = functools.partial(
        pl.kernel,
        mesh=mesh,
        out_type=jax.ShapeDtypeStruct((B, L, HIDDEN), jnp.float32),
        scratch_types=[
            pltpu.VMEM((L, HIDDEN), jnp.float32),
            pltpu.VMEM((_RPC, L), jnp.int32),
            pltpu.VMEM((_SUB * L, 128), jnp.float32),
            pltpu.VMEM((_SUB, L, HIDDEN), jnp.float32),
            pltpu.SemaphoreType.DMA,
        ],
        compiler_params=pltpu.CompilerParams(use_tc_tiling_on_sc=True),
    )(_sc_emb_body)
    return f(x, tok_wide, pos_table)


_RPB = 1000


def _repack_body(t_ref, o_ref):
    o_ref[:, :HIDDEN] = t_ref[...]


def _tc_repack(token_table):
    return pl.pallas_call(
        _repack_body,
        grid=(VOCAB // _RPB,),
        in_specs=[pl.BlockSpec((_RPB, HIDDEN), lambda i: (i, 0))],
        out_specs=pl.BlockSpec((_RPB, 128), lambda i: (i, 0)),
        out_shape=jax.ShapeDtypeStruct((VOCAB, 128), jnp.float32),
    )(token_table)


_T_BB = 256
_T_LB = 8


def _tr_body(i_ref, o_ref):
    r = lax.broadcasted_iota(jnp.int32, (HIDDEN, HIDDEN), 0)
    c = lax.broadcasted_iota(jnp.int32, (HIDDEN, HIDDEN), 1)
    ident = (r == c).astype(jnp.float32)
    for l in range(_T_LB):
        o_ref[l] = lax.dot_general(
            ident, i_ref[:, l, :],
            (((1,), (1,)), ((), ())),
            preferred_element_type=jnp.float32,
            precision=lax.Precision.HIGHEST,
        )


def _tc_transpose(emb_blh):
    return pl.pallas_call(
        _tr_body,
        grid=(L // _T_LB, B // _T_BB),
        in_specs=[pl.BlockSpec((_T_BB, _T_LB, HIDDEN), lambda l, b: (b, l, 0))],
        out_specs=pl.BlockSpec((_T_LB, HIDDEN, _T_BB), lambda l, b: (l, 0, b)),
        out_shape=jax.ShapeDtypeStruct((L, HIDDEN, B), jnp.float32),
    )(emb_blh)


_MASK_BB = 128


def _mask_body(xt_ref, o_ref):
    m = (xt_ref[...] > 0).astype(jnp.int8)

    def body(i, c):
        o_ref[i] = m
        return c

    lax.fori_loop(0, L, body, 0)


def _tc_mask(xt):
    return pl.pallas_call(
        _mask_body,
        grid=(B // _MASK_BB,),
        in_specs=[pl.BlockSpec((L, _MASK_BB), lambda i: (0, i))],
        out_specs=pl.BlockSpec((L, L, _MASK_BB), lambda i: (0, 0, i)),
        out_shape=jax.ShapeDtypeStruct((L, L, B), jnp.int8),
    )(xt)


def kernel(x, token_table, pos_table):
    tok_wide = _tc_repack(token_table)
    emb_blh = _sc_emb(x, tok_wide, pos_table)
    emb = jnp.transpose(_tc_transpose(emb_blh), (2, 0, 1))
    m8 = _tc_mask(jnp.transpose(x))
    mask = jnp.transpose(m8, (2, 0, 1))[:, None, :, :].astype(jnp.bool_)
    return emb, mask
